# baseline (speedup 1.0000x reference)
import functools

import jax
import jax.numpy as jnp
from jax import lax
from jax.experimental import pallas as pl
from jax.experimental.pallas import tpu as pltpu
from jax.experimental.pallas import tpu_sc as plsc

_N = 10000
_E = 320000
_D = 128
_NG = 64

_NCORE = 2
_NSUB = 16
_NW = _NCORE * _NSUB

_CHUNK = 128
_CPT = 80
_EPT = _CHUNK * _CPT
_EP = _EPT * _NW
_NPAD = 10112
_ZRPT = _NPAD // _NSUB
_DEGW = 16

_R = 1000
_GRID = _N // _R


def _sc_mesh():
    return plsc.VectorSubcoreMesh(
        core_axis_name="c", subcore_axis_name="s",
        num_cores=_NCORE, num_subcores=_NSUB)


@functools.partial(
    pl.kernel,
    out_type=jax.ShapeDtypeStruct((_NCORE, _NPAD, _D), jnp.float32),
    mesh=_sc_mesh(),
    scratch_types=[
        pltpu.VMEM((_CPT, _CHUNK), jnp.int32),
        pltpu.VMEM((_CHUNK, _D), jnp.float32),
        pltpu.VMEM_SHARED((_NPAD, _D), jnp.float32),
        pltpu.SemaphoreType.DMA,
    ],
)
def _sc_deg(dst_hbm, zeros_hbm, ones_hbm, out_hbm, dst_v, ones_v, acc, sem):
    cid = lax.axis_index("c")
    sid = lax.axis_index("s")
    wid = sid * _NCORE + cid
    pltpu.sync_copy(zeros_hbm.at[pl.ds(sid * _ZRPT, _ZRPT)],
                    acc.at[pl.ds(sid * _ZRPT, _ZRPT)])
    pltpu.sync_copy(ones_hbm, ones_v)
    pltpu.sync_copy(dst_hbm.at[wid], dst_v)
    plsc.subcore_barrier()

    @pl.loop(0, _CPT // 8)
    def _(g):
        for u in range(8):
            pltpu.async_copy(ones_v, acc.at[dst_v.at[8 * g + u]], sem,
                             add=True)
        for u in range(8):
            pltpu.make_async_copy(ones_v, acc.at[dst_v.at[0]], sem).wait()

    plsc.subcore_barrier()
    pltpu.sync_copy(acc.at[pl.ds(sid * _ZRPT, _ZRPT)],
                    out_hbm.at[cid, pl.ds(sid * _ZRPT, _ZRPT)])


@functools.partial(
    pl.kernel,
    out_type=jax.ShapeDtypeStruct((_NCORE, _NPAD, _D), jnp.float32),
    mesh=_sc_mesh(),
    scratch_types=[
        pltpu.VMEM((_CPT, _CHUNK), jnp.int32),
        pltpu.VMEM((_CPT // 2, _CHUNK), jnp.int32),
        pltpu.VMEM((_CHUNK, _D), jnp.float32),
        pltpu.VMEM((_CHUNK, _D), jnp.float32),
        pltpu.VMEM_SHARED((_NPAD, _D), jnp.float32),
        pltpu.SemaphoreType.DMA,
        pltpu.SemaphoreType.DMA,
        pltpu.SemaphoreType.DMA,
        pltpu.SemaphoreType.DMA,
    ],
)
def _sc_msg(z_hbm, src_hbm, dst_hbm, zeros_hbm, out_hbm,
            src_v, dst_v, rows_a, rows_b, acc,
            gsem_a, gsem_b, ssem_a, ssem_b):
    cid = lax.axis_index("c")
    sid = lax.axis_index("s")
    wid = sid * _NCORE + cid
    half = _CPT // 2
    pltpu.sync_copy(zeros_hbm.at[pl.ds(sid * _ZRPT, _ZRPT)],
                    acc.at[pl.ds(sid * _ZRPT, _ZRPT)])
    pltpu.sync_copy(src_hbm.at[wid], src_v)
    pltpu.sync_copy(dst_hbm.at[wid, pl.ds(0, half)], dst_v)

    def g_start(i, buf, sem):
        pltpu.async_copy(z_hbm.at[src_v.at[i]], buf, sem)

    def g_drain(buf, sem):
        pltpu.make_async_copy(z_hbm.at[pl.ds(0, _CHUNK)], buf, sem).wait()

    def s_start(k, buf, sem):
        pltpu.async_copy(buf, acc.at[dst_v.at[k]], sem, add=True)

    def s_drain(buf, sem):
        pltpu.make_async_copy(buf, acc.at[dst_v.at[0]], sem).wait()

    g_start(0, rows_a, gsem_a)
    g_start(1, rows_b, gsem_b)
    plsc.subcore_barrier()

    for phase in range(2):
        base_c = phase * half

        @pl.loop(0, half // 2)
        def _(j):
            i = base_c + 2 * j
            k = 2 * j
            g_drain(rows_a, gsem_a)
            s_start(k, rows_a, ssem_a)
            g_drain(rows_b, gsem_b)
            s_start(k + 1, rows_b, ssem_b)
            s_drain(rows_a, ssem_a)

            @pl.when(i + 2 < _CPT)
            def _():
                g_start(i + 2, rows_a, gsem_a)

            s_drain(rows_b, ssem_b)

            @pl.when(i + 3 < _CPT)
            def _():
                g_start(i + 3, rows_b, gsem_b)

        if phase == 0:
            pltpu.sync_copy(dst_hbm.at[wid, pl.ds(half, half)], dst_v)

    plsc.subcore_barrier()
    pltpu.sync_copy(acc.at[pl.ds(sid * _ZRPT, _ZRPT)],
                    out_hbm.at[cid, pl.ds(sid * _ZRPT, _ZRPT)])


def _dot(a, b):
    return lax.dot_general(a, b, (((1,), (0,)), ((), ())),
                           preferred_element_type=jnp.float32,
                           precision=lax.Precision.HIGHEST)


def _mm_body(x_ref, w_ref, o_ref):
    o_ref[...] = _dot(x_ref[...], w_ref[...])


_tc_mm = pl.pallas_call(
    _mm_body,
    grid=(_GRID,),
    in_specs=[pl.BlockSpec((_R, _D), lambda i: (i, 0)),
              pl.BlockSpec((_D, _D), lambda i: (0, 0))],
    out_specs=pl.BlockSpec((_R, _D), lambda i: (i, 0)),
    out_shape=jax.ShapeDtypeStruct((_N, _D), jnp.float32),
)


def _prep_body(d0_ref, d1_ref, y_ref, dis_ref, z_ref):
    deg = d0_ref[...][:, 0:1] + d1_ref[...][:, 0:1] + 1.0
    dis = lax.rsqrt(deg)
    dis_ref[...] = jnp.broadcast_to(dis, (_R, _DEGW))
    z_ref[...] = y_ref[...] * dis


_tc_prep = pl.pallas_call(
    _prep_body,
    grid=(_GRID,),
    in_specs=[pl.BlockSpec((_R, _D), lambda i: (i, 0)),
              pl.BlockSpec((_R, _D), lambda i: (i, 0)),
              pl.BlockSpec((_R, _D), lambda i: (i, 0))],
    out_specs=[pl.BlockSpec((_R, _DEGW), lambda i: (i, 0)),
               pl.BlockSpec((_R, _D), lambda i: (i, 0))],
    out_shape=[jax.ShapeDtypeStruct((_N, _DEGW), jnp.float32),
               jax.ShapeDtypeStruct((_N, _D), jnp.float32)],
)


def _elu(h):
    return jnp.where(h > 0.0, h, jnp.exp(jnp.minimum(h, 0.0)) - 1.0)


def _layer_body(a0_ref, a1_ref, z_ref, dis_ref, b_ref, w_ref, o_ref):
    dis = dis_ref[...][:, 0:1]
    h = dis * (a0_ref[...] + a1_ref[...] + z_ref[...]) + b_ref[...]
    u = _elu(h)
    o_ref[...] = dis * _dot(u, w_ref[...])


_tc_layer = pl.pallas_call(
    _layer_body,
    grid=(_GRID,),
    in_specs=[pl.BlockSpec((_R, _D), lambda i: (i, 0)),
              pl.BlockSpec((_R, _D), lambda i: (i, 0)),
              pl.BlockSpec((_R, _D), lambda i: (i, 0)),
              pl.BlockSpec((_R, _DEGW), lambda i: (i, 0)),
              pl.BlockSpec((1, _D), lambda i: (0, 0)),
              pl.BlockSpec((_D, _D), lambda i: (0, 0))],
    out_specs=pl.BlockSpec((_R, _D), lambda i: (i, 0)),
    out_shape=jax.ShapeDtypeStruct((_N, _D), jnp.float32),
)


def _final_body(a0_ref, a1_ref, z_ref, dis_ref, b_ref, bat_ref,
                wc_ref, bc_ref, wr_ref, br_ref,
                pooled_ref, cls_ref, reg_ref, ps_ref, cnt_ref):
    i = pl.program_id(0)

    @pl.when(i == 0)
    def _():
        ps_ref[...] = jnp.zeros_like(ps_ref)
        cnt_ref[...] = jnp.zeros_like(cnt_ref)

    dis = dis_ref[...][:, 0:1]
    h = dis * (a0_ref[...] + a1_ref[...] + z_ref[...]) + b_ref[...]
    gids = lax.broadcasted_iota(jnp.int32, (_R, _NG), 1).astype(jnp.float32)
    oh = (bat_ref[...][:, 0:1] == gids).astype(jnp.float32)
    ps_ref[...] += lax.dot_general(oh, h, (((0,), (0,)), ((), ())),
                                   preferred_element_type=jnp.float32,
                                   precision=lax.Precision.HIGHEST)
    ones = jnp.ones((_R, _DEGW), jnp.float32)
    cnt_ref[...] += lax.dot_general(oh, ones, (((0,), (0,)), ((), ())),
                                    preferred_element_type=jnp.float32,
                                    precision=lax.Precision.HIGHEST)

    @pl.when(i == _GRID - 1)
    def _():
        pooled = ps_ref[...] / jnp.maximum(cnt_ref[...][:, 0:1], 1.0)
        pooled_ref[...] = pooled
        cls_ref[...] = _dot(pooled, wc_ref[...]) + bc_ref[...]
        reg_ref[...] = _dot(pooled, wr_ref[...]) + br_ref[...]


_tc_final = pl.pallas_call(
    _final_body,
    grid=(_GRID,),
    in_specs=[pl.BlockSpec((_R, _D), lambda i: (i, 0)),
              pl.BlockSpec((_R, _D), lambda i: (i, 0)),
              pl.BlockSpec((_R, _D), lambda i: (i, 0)),
              pl.BlockSpec((_R, _DEGW), lambda i: (i, 0)),
              pl.BlockSpec((1, _D), lambda i: (0, 0)),
              pl.BlockSpec((_R, _DEGW), lambda i: (i, 0)),
              pl.BlockSpec((_D, 16), lambda i: (0, 0)),
              pl.BlockSpec((1, 16), lambda i: (0, 0)),
              pl.BlockSpec((_D, 4), lambda i: (0, 0)),
              pl.BlockSpec((1, 4), lambda i: (0, 0))],
    out_specs=[pl.BlockSpec((_NG, _D), lambda i: (0, 0)),
               pl.BlockSpec((_NG, 16), lambda i: (0, 0)),
               pl.BlockSpec((_NG, 4), lambda i: (0, 0))],
    out_shape=[jax.ShapeDtypeStruct((_NG, _D), jnp.float32),
               jax.ShapeDtypeStruct((_NG, 16), jnp.float32),
               jax.ShapeDtypeStruct((_NG, 4), jnp.float32)],
    scratch_shapes=[pltpu.VMEM((_NG, _D), jnp.float32),
                    pltpu.VMEM((_NG, _DEGW), jnp.float32)],
)


def kernel(x, edge_index, batch, W1, b1, W2, b2, W3, b3, Wc, bc, Wr, br):
    src = edge_index[0]
    dst = edge_index[1]
    pad = _EP - _E
    ar = jnp.arange(pad, dtype=jnp.int32)
    srcp = jnp.concatenate([src, (ar * 8) % _N]).reshape(_NW, _CPT, _CHUNK)
    dstp = jnp.concatenate([dst, _N + (ar % (_NPAD - _N))]
                           ).reshape(_NW, _CPT, _CHUNK)

    zeros128 = jnp.zeros((_NPAD, _D), jnp.float32)
    ones_c = jnp.ones((_CHUNK, _D), jnp.float32)
    bat16 = jnp.broadcast_to(
        batch.astype(jnp.float32)[:, None], (_N, _DEGW))

    degp = _sc_deg(dstp, zeros128, ones_c)
    y1 = _tc_mm(x, W1)
    dis16, z1 = _tc_prep(degp[0, :_N], degp[1, :_N], y1)

    a1 = _sc_msg(z1, srcp, dstp, zeros128)
    z2 = _tc_layer(a1[0, :_N], a1[1, :_N], z1, dis16, b1.reshape(1, _D), W2)
    a2 = _sc_msg(z2, srcp, dstp, zeros128)
    z3 = _tc_layer(a2[0, :_N], a2[1, :_N], z2, dis16, b2.reshape(1, _D), W3)
    a3 = _sc_msg(z3, srcp, dstp, zeros128)

    pooled, cls_o, reg_o = _tc_final(
        a3[0, :_N], a3[1, :_N], z3, dis16, b3.reshape(1, _D), bat16,
        Wc, bc.reshape(1, 16), Wr, br.reshape(1, 4))
    return pooled, cls_o, reg_o


# speedup vs baseline: 1.2280x; 1.2280x over previous
import functools

import jax
import jax.numpy as jnp
from jax import lax
from jax.experimental import pallas as pl
from jax.experimental.pallas import tpu as pltpu
from jax.experimental.pallas import tpu_sc as plsc

_N = 10000
_E = 320000
_D = 128
_NG = 64

_NCORE = 2
_NSUB = 16
_NW = _NCORE * _NSUB

_CHUNK = 128
_CPT = 80
_EPT = _CHUNK * _CPT
_EP = _EPT * _NW
_NPAD = 10112
_ZRPT = _NPAD // _NSUB
_DEGW = 16

_R = 1000
_GRID = _N // _R


def _sc_mesh():
    return plsc.VectorSubcoreMesh(
        core_axis_name="c", subcore_axis_name="s",
        num_cores=_NCORE, num_subcores=_NSUB)


@functools.partial(
    pl.kernel,
    out_type=jax.ShapeDtypeStruct((_NCORE, _NPAD, _D), jnp.float32),
    mesh=_sc_mesh(),
    scratch_types=[
        pltpu.VMEM((_CPT, _CHUNK), jnp.int32),
        pltpu.VMEM((_CHUNK, _D), jnp.float32),
        pltpu.VMEM_SHARED((_NPAD, _D), jnp.float32),
        pltpu.SemaphoreType.DMA,
    ],
)
def _sc_deg(dst_hbm, zeros_hbm, ones_hbm, out_hbm, dst_v, ones_v, acc, sem):
    cid = lax.axis_index("c")
    sid = lax.axis_index("s")
    wid = sid * _NCORE + cid
    pltpu.sync_copy(zeros_hbm.at[pl.ds(sid * _ZRPT, _ZRPT)],
                    acc.at[pl.ds(sid * _ZRPT, _ZRPT)])
    pltpu.sync_copy(ones_hbm, ones_v)
    pltpu.sync_copy(dst_hbm.at[wid], dst_v)
    plsc.subcore_barrier()

    @pl.loop(0, _CPT // 8)
    def _(g):
        for u in range(8):
            pltpu.async_copy(ones_v, acc.at[dst_v.at[8 * g + u]], sem,
                             add=True)
        for u in range(8):
            pltpu.make_async_copy(ones_v, acc.at[dst_v.at[0]], sem).wait()

    plsc.subcore_barrier()
    pltpu.sync_copy(acc.at[pl.ds(sid * _ZRPT, _ZRPT)],
                    out_hbm.at[cid, pl.ds(sid * _ZRPT, _ZRPT)])


@functools.partial(
    pl.kernel,
    out_type=jax.ShapeDtypeStruct((_NCORE, _NPAD, _D), jnp.float32),
    mesh=_sc_mesh(),
    scratch_types=[
        pltpu.VMEM((_CPT, _CHUNK), jnp.int32),
        pltpu.VMEM((_CPT // 2, _CHUNK), jnp.int32),
        pltpu.VMEM((_CHUNK, _D), jnp.float32),
        pltpu.VMEM((_CHUNK, _D), jnp.float32),
        pltpu.VMEM_SHARED((_NPAD, _D), jnp.float32),
        pltpu.SemaphoreType.DMA,
        pltpu.SemaphoreType.DMA,
        pltpu.SemaphoreType.DMA,
    ],
)
def _sc_msg(z_hbm, src_hbm, dst_hbm, zeros_hbm, out_hbm,
            src_v, dst_v, rows_a, rows_b, acc, sem_a, sem_b, psem):
    cid = lax.axis_index("c")
    sid = lax.axis_index("s")
    wid = sid * _NCORE + cid
    half = _CPT // 2
    pltpu.async_copy(zeros_hbm.at[pl.ds(sid * _ZRPT, _ZRPT)],
                     acc.at[pl.ds(sid * _ZRPT, _ZRPT)], psem)
    pltpu.async_copy(src_hbm.at[wid], src_v, sem_a)
    pltpu.async_copy(dst_hbm.at[wid, pl.ds(0, half)], dst_v, sem_b)
    pltpu.make_async_copy(src_hbm.at[wid], src_v, sem_a).wait()
    pltpu.make_async_copy(dst_hbm.at[wid, pl.ds(0, half)], dst_v,
                          sem_b).wait()

    def start(i, buf, sem):
        pltpu.async_copy(z_hbm.at[src_v.at[i]], buf, sem)

    def drain(buf, sem):
        pltpu.make_async_copy(z_hbm.at[pl.ds(0, _CHUNK)], buf, sem).wait()

    start(0, rows_a, sem_a)
    start(1, rows_b, sem_b)
    pltpu.make_async_copy(zeros_hbm.at[pl.ds(sid * _ZRPT, _ZRPT)],
                          acc.at[pl.ds(sid * _ZRPT, _ZRPT)], psem).wait()
    plsc.subcore_barrier()

    for phase in range(2):
        base_c = phase * half

        @pl.loop(0, half // 2)
        def _(j):
            i = base_c + 2 * j
            k = 2 * j
            drain(rows_a, sem_a)
            pltpu.sync_copy(rows_a, acc.at[dst_v.at[k]], add=True)

            @pl.when(i + 2 < _CPT)
            def _():
                start(i + 2, rows_a, sem_a)

            drain(rows_b, sem_b)
            pltpu.sync_copy(rows_b, acc.at[dst_v.at[k + 1]], add=True)

            @pl.when(i + 3 < _CPT)
            def _():
                start(i + 3, rows_b, sem_b)

        if phase == 0:
            pltpu.sync_copy(dst_hbm.at[wid, pl.ds(half, half)], dst_v)

    plsc.subcore_barrier()
    pltpu.sync_copy(acc.at[pl.ds(sid * _ZRPT, _ZRPT)],
                    out_hbm.at[cid, pl.ds(sid * _ZRPT, _ZRPT)])


def _dot(a, b):
    return lax.dot_general(a, b, (((1,), (0,)), ((), ())),
                           preferred_element_type=jnp.float32,
                           precision=lax.Precision.HIGHEST)


def _mm_body(x_ref, w_ref, o_ref):
    o_ref[...] = _dot(x_ref[...], w_ref[...])


_tc_mm = pl.pallas_call(
    _mm_body,
    grid=(_GRID,),
    in_specs=[pl.BlockSpec((_R, _D), lambda i: (i, 0)),
              pl.BlockSpec((_D, _D), lambda i: (0, 0))],
    out_specs=pl.BlockSpec((_R, _D), lambda i: (i, 0)),
    out_shape=jax.ShapeDtypeStruct((_N, _D), jnp.float32),
)


def _prep_body(d0_ref, d1_ref, y_ref, dis_ref, z_ref):
    deg = d0_ref[...][:, 0:1] + d1_ref[...][:, 0:1] + 1.0
    dis = lax.rsqrt(deg)
    dis_ref[...] = jnp.broadcast_to(dis, (_R, _DEGW))
    z_ref[...] = y_ref[...] * dis


_tc_prep = pl.pallas_call(
    _prep_body,
    grid=(_GRID,),
    in_specs=[pl.BlockSpec((_R, _D), lambda i: (i, 0)),
              pl.BlockSpec((_R, _D), lambda i: (i, 0)),
              pl.BlockSpec((_R, _D), lambda i: (i, 0))],
    out_specs=[pl.BlockSpec((_R, _DEGW), lambda i: (i, 0)),
               pl.BlockSpec((_R, _D), lambda i: (i, 0))],
    out_shape=[jax.ShapeDtypeStruct((_N, _DEGW), jnp.float32),
               jax.ShapeDtypeStruct((_N, _D), jnp.float32)],
)


def _elu(h):
    return jnp.where(h > 0.0, h, jnp.exp(jnp.minimum(h, 0.0)) - 1.0)


def _layer_body(a0_ref, a1_ref, z_ref, dis_ref, b_ref, w_ref, o_ref):
    dis = dis_ref[...][:, 0:1]
    h = dis * (a0_ref[...] + a1_ref[...] + z_ref[...]) + b_ref[...]
    u = _elu(h)
    o_ref[...] = dis * _dot(u, w_ref[...])


_tc_layer = pl.pallas_call(
    _layer_body,
    grid=(_GRID,),
    in_specs=[pl.BlockSpec((_R, _D), lambda i: (i, 0)),
              pl.BlockSpec((_R, _D), lambda i: (i, 0)),
              pl.BlockSpec((_R, _D), lambda i: (i, 0)),
              pl.BlockSpec((_R, _DEGW), lambda i: (i, 0)),
              pl.BlockSpec((1, _D), lambda i: (0, 0)),
              pl.BlockSpec((_D, _D), lambda i: (0, 0))],
    out_specs=pl.BlockSpec((_R, _D), lambda i: (i, 0)),
    out_shape=jax.ShapeDtypeStruct((_N, _D), jnp.float32),
)


def _final_body(a0_ref, a1_ref, z_ref, dis_ref, b_ref, bat_ref,
                wc_ref, bc_ref, wr_ref, br_ref,
                pooled_ref, cls_ref, reg_ref, ps_ref, cnt_ref):
    i = pl.program_id(0)

    @pl.when(i == 0)
    def _():
        ps_ref[...] = jnp.zeros_like(ps_ref)
        cnt_ref[...] = jnp.zeros_like(cnt_ref)

    dis = dis_ref[...][:, 0:1]
    h = dis * (a0_ref[...] + a1_ref[...] + z_ref[...]) + b_ref[...]
    gids = lax.broadcasted_iota(jnp.int32, (_R, _NG), 1).astype(jnp.float32)
    oh = (bat_ref[...][:, 0:1] == gids).astype(jnp.float32)
    ps_ref[...] += lax.dot_general(oh, h, (((0,), (0,)), ((), ())),
                                   preferred_element_type=jnp.float32,
                                   precision=lax.Precision.HIGHEST)
    ones = jnp.ones((_R, _DEGW), jnp.float32)
    cnt_ref[...] += lax.dot_general(oh, ones, (((0,), (0,)), ((), ())),
                                    preferred_element_type=jnp.float32,
                                    precision=lax.Precision.HIGHEST)

    @pl.when(i == _GRID - 1)
    def _():
        pooled = ps_ref[...] / jnp.maximum(cnt_ref[...][:, 0:1], 1.0)
        pooled_ref[...] = pooled
        cls_ref[...] = _dot(pooled, wc_ref[...]) + bc_ref[...]
        reg_ref[...] = _dot(pooled, wr_ref[...]) + br_ref[...]


_tc_final = pl.pallas_call(
    _final_body,
    grid=(_GRID,),
    in_specs=[pl.BlockSpec((_R, _D), lambda i: (i, 0)),
              pl.BlockSpec((_R, _D), lambda i: (i, 0)),
              pl.BlockSpec((_R, _D), lambda i: (i, 0)),
              pl.BlockSpec((_R, _DEGW), lambda i: (i, 0)),
              pl.BlockSpec((1, _D), lambda i: (0, 0)),
              pl.BlockSpec((_R, _DEGW), lambda i: (i, 0)),
              pl.BlockSpec((_D, 16), lambda i: (0, 0)),
              pl.BlockSpec((1, 16), lambda i: (0, 0)),
              pl.BlockSpec((_D, 4), lambda i: (0, 0)),
              pl.BlockSpec((1, 4), lambda i: (0, 0))],
    out_specs=[pl.BlockSpec((_NG, _D), lambda i: (0, 0)),
               pl.BlockSpec((_NG, 16), lambda i: (0, 0)),
               pl.BlockSpec((_NG, 4), lambda i: (0, 0))],
    out_shape=[jax.ShapeDtypeStruct((_NG, _D), jnp.float32),
               jax.ShapeDtypeStruct((_NG, 16), jnp.float32),
               jax.ShapeDtypeStruct((_NG, 4), jnp.float32)],
    scratch_shapes=[pltpu.VMEM((_NG, _D), jnp.float32),
                    pltpu.VMEM((_NG, _DEGW), jnp.float32)],
)


def kernel(x, edge_index, batch, W1, b1, W2, b2, W3, b3, Wc, bc, Wr, br):
    src = edge_index[0]
    dst = edge_index[1]
    pad = _EP - _E
    ar = jnp.arange(pad, dtype=jnp.int32)
    srcp = jnp.concatenate([src, (ar * 8) % _N]).reshape(_NW, _CPT, _CHUNK)
    dstp = jnp.concatenate([dst, _N + (ar % (_NPAD - _N))]
                           ).reshape(_NW, _CPT, _CHUNK)

    zeros128 = jnp.zeros((_NPAD, _D), jnp.float32)
    ones_c = jnp.ones((_CHUNK, _D), jnp.float32)
    bat16 = jnp.broadcast_to(
        batch.astype(jnp.float32)[:, None], (_N, _DEGW))

    degp = _sc_deg(dstp, zeros128, ones_c)
    y1 = _tc_mm(x, W1)
    dis16, z1 = _tc_prep(degp[0, :_N], degp[1, :_N], y1)

    a1 = _sc_msg(z1, srcp, dstp, zeros128)
    z2 = _tc_layer(a1[0, :_N], a1[1, :_N], z1, dis16, b1.reshape(1, _D), W2)
    a2 = _sc_msg(z2, srcp, dstp, zeros128)
    z3 = _tc_layer(a2[0, :_N], a2[1, :_N], z2, dis16, b2.reshape(1, _D), W3)
    a3 = _sc_msg(z3, srcp, dstp, zeros128)

    pooled, cls_o, reg_o = _tc_final(
        a3[0, :_N], a3[1, :_N], z3, dis16, b3.reshape(1, _D), bat16,
        Wc, bc.reshape(1, 16), Wr, br.reshape(1, 4))
    return pooled, cls_o, reg_o


# speedup vs baseline: 1.2435x; 1.0126x over previous
import functools

import jax
import jax.numpy as jnp
from jax import lax
from jax.experimental import pallas as pl
from jax.experimental.pallas import tpu as pltpu
from jax.experimental.pallas import tpu_sc as plsc

_N = 10000
_E = 320000
_D = 128
_NG = 64

_NCORE = 2
_NSUB = 16
_NW = _NCORE * _NSUB

_CHUNK = 128
_CPT = 80
_EPT = _CHUNK * _CPT
_EP = _EPT * _NW
_NPAD = 10112
_ZRPT = _NPAD // _NSUB
_DEGW = 16

_R = 1000
_GRID = _N // _R


def _sc_mesh():
    return plsc.VectorSubcoreMesh(
        core_axis_name="c", subcore_axis_name="s",
        num_cores=_NCORE, num_subcores=_NSUB)


@functools.partial(
    pl.kernel,
    out_type=jax.ShapeDtypeStruct((_NCORE, _NPAD, _D), jnp.float32),
    mesh=_sc_mesh(),
    scratch_types=[
        pltpu.VMEM((_CPT, _CHUNK), jnp.int32),
        pltpu.VMEM((_CHUNK, _D), jnp.float32),
        pltpu.VMEM_SHARED((_NPAD, _D), jnp.float32),
        pltpu.SemaphoreType.DMA,
        pltpu.SemaphoreType.DMA,
    ],
)
def _sc_deg(dst_hbm, zeros_hbm, ones_hbm, out_hbm, dst_v, ones_v, acc,
            sem, psem):
    cid = lax.axis_index("c")
    sid = lax.axis_index("s")
    wid = sid * _NCORE + cid
    pltpu.async_copy(zeros_hbm.at[pl.ds(sid * _ZRPT, _ZRPT)],
                     acc.at[pl.ds(sid * _ZRPT, _ZRPT)], psem)
    pltpu.async_copy(ones_hbm, ones_v, sem)
    pltpu.async_copy(dst_hbm.at[wid], dst_v, sem)
    pltpu.make_async_copy(ones_hbm, ones_v, sem).wait()
    pltpu.make_async_copy(dst_hbm.at[wid], dst_v, sem).wait()
    pltpu.make_async_copy(zeros_hbm.at[pl.ds(sid * _ZRPT, _ZRPT)],
                          acc.at[pl.ds(sid * _ZRPT, _ZRPT)], psem).wait()
    plsc.subcore_barrier()

    @pl.loop(0, _CPT // 8)
    def _(g):
        for u in range(8):
            pltpu.async_copy(ones_v, acc.at[dst_v.at[8 * g + u]], sem,
                             add=True)
        for u in range(8):
            pltpu.make_async_copy(ones_v, acc.at[dst_v.at[0]], sem).wait()

    plsc.subcore_barrier()
    pltpu.sync_copy(acc.at[pl.ds(sid * _ZRPT, _ZRPT)],
                    out_hbm.at[cid, pl.ds(sid * _ZRPT, _ZRPT)])


@functools.partial(
    pl.kernel,
    out_type=jax.ShapeDtypeStruct((_NCORE, _NPAD, _D), jnp.float32),
    mesh=_sc_mesh(),
    scratch_types=[
        pltpu.VMEM((_CPT, _CHUNK), jnp.int32),
        pltpu.VMEM((_CPT // 2, _CHUNK), jnp.int32),
        pltpu.VMEM((_CHUNK, _D), jnp.float32),
        pltpu.VMEM((_CHUNK, _D), jnp.float32),
        pltpu.VMEM_SHARED((_NPAD, _D), jnp.float32),
        pltpu.SemaphoreType.DMA,
        pltpu.SemaphoreType.DMA,
        pltpu.SemaphoreType.DMA,
    ],
)
def _sc_msg(z_hbm, src_hbm, dst_hbm, zeros_hbm, out_hbm,
            src_v, dst_v, rows_a, rows_b, acc, sem_a, sem_b, psem):
    cid = lax.axis_index("c")
    sid = lax.axis_index("s")
    wid = sid * _NCORE + cid
    half = _CPT // 2
    pltpu.async_copy(zeros_hbm.at[pl.ds(sid * _ZRPT, _ZRPT)],
                     acc.at[pl.ds(sid * _ZRPT, _ZRPT)], psem)
    pltpu.async_copy(src_hbm.at[wid], src_v, sem_a)
    pltpu.async_copy(dst_hbm.at[wid, pl.ds(0, half)], dst_v, sem_b)
    pltpu.make_async_copy(src_hbm.at[wid], src_v, sem_a).wait()
    pltpu.make_async_copy(dst_hbm.at[wid, pl.ds(0, half)], dst_v,
                          sem_b).wait()

    def start(i, buf, sem):
        pltpu.async_copy(z_hbm.at[src_v.at[i]], buf, sem)

    def drain(buf, sem):
        pltpu.make_async_copy(z_hbm.at[pl.ds(0, _CHUNK)], buf, sem).wait()

    start(0, rows_a, sem_a)
    start(1, rows_b, sem_b)
    pltpu.make_async_copy(zeros_hbm.at[pl.ds(sid * _ZRPT, _ZRPT)],
                          acc.at[pl.ds(sid * _ZRPT, _ZRPT)], psem).wait()
    plsc.subcore_barrier()

    for phase in range(2):
        base_c = phase * half

        @pl.loop(0, half // 2)
        def _(j):
            i = base_c + 2 * j
            k = 2 * j
            drain(rows_a, sem_a)
            pltpu.sync_copy(rows_a, acc.at[dst_v.at[k]], add=True)

            @pl.when(i + 2 < _CPT)
            def _():
                start(i + 2, rows_a, sem_a)

            drain(rows_b, sem_b)
            pltpu.sync_copy(rows_b, acc.at[dst_v.at[k + 1]], add=True)

            @pl.when(i + 3 < _CPT)
            def _():
                start(i + 3, rows_b, sem_b)

        if phase == 0:
            pltpu.sync_copy(dst_hbm.at[wid, pl.ds(half, half)], dst_v)

    plsc.subcore_barrier()
    pltpu.sync_copy(acc.at[pl.ds(sid * _ZRPT, _ZRPT)],
                    out_hbm.at[cid, pl.ds(sid * _ZRPT, _ZRPT)])


def _dot(a, b):
    return lax.dot_general(a, b, (((1,), (0,)), ((), ())),
                           preferred_element_type=jnp.float32,
                           precision=lax.Precision.DEFAULT)


def _mm_body(x_ref, w_ref, o_ref):
    o_ref[...] = _dot(x_ref[...], w_ref[...])


_tc_mm = pl.pallas_call(
    _mm_body,
    grid=(_GRID,),
    in_specs=[pl.BlockSpec((_R, _D), lambda i: (i, 0)),
              pl.BlockSpec((_D, _D), lambda i: (0, 0))],
    out_specs=pl.BlockSpec((_R, _D), lambda i: (i, 0)),
    out_shape=jax.ShapeDtypeStruct((_N, _D), jnp.float32),
)


def _prep_body(d0_ref, d1_ref, y_ref, dis_ref, z_ref):
    deg = d0_ref[...][:, 0:1] + d1_ref[...][:, 0:1] + 1.0
    dis = lax.rsqrt(deg)
    dis_ref[...] = jnp.broadcast_to(dis, (_R, _DEGW))
    z_ref[...] = y_ref[...] * dis


_tc_prep = pl.pallas_call(
    _prep_body,
    grid=(_GRID,),
    in_specs=[pl.BlockSpec((_R, _D), lambda i: (i, 0)),
              pl.BlockSpec((_R, _D), lambda i: (i, 0)),
              pl.BlockSpec((_R, _D), lambda i: (i, 0))],
    out_specs=[pl.BlockSpec((_R, _DEGW), lambda i: (i, 0)),
               pl.BlockSpec((_R, _D), lambda i: (i, 0))],
    out_shape=[jax.ShapeDtypeStruct((_N, _DEGW), jnp.float32),
               jax.ShapeDtypeStruct((_N, _D), jnp.float32)],
)


def _elu(h):
    return jnp.where(h > 0.0, h, jnp.exp(jnp.minimum(h, 0.0)) - 1.0)


def _layer_body(a0_ref, a1_ref, z_ref, dis_ref, b_ref, w_ref, o_ref):
    dis = dis_ref[...][:, 0:1]
    h = dis * (a0_ref[...] + a1_ref[...] + z_ref[...]) + b_ref[...]
    u = _elu(h)
    o_ref[...] = dis * _dot(u, w_ref[...])


_tc_layer = pl.pallas_call(
    _layer_body,
    grid=(_GRID,),
    in_specs=[pl.BlockSpec((_R, _D), lambda i: (i, 0)),
              pl.BlockSpec((_R, _D), lambda i: (i, 0)),
              pl.BlockSpec((_R, _D), lambda i: (i, 0)),
              pl.BlockSpec((_R, _DEGW), lambda i: (i, 0)),
              pl.BlockSpec((1, _D), lambda i: (0, 0)),
              pl.BlockSpec((_D, _D), lambda i: (0, 0))],
    out_specs=pl.BlockSpec((_R, _D), lambda i: (i, 0)),
    out_shape=jax.ShapeDtypeStruct((_N, _D), jnp.float32),
)


def _final_body(a0_ref, a1_ref, z_ref, dis_ref, b_ref, bat_ref,
                wc_ref, bc_ref, wr_ref, br_ref,
                pooled_ref, cls_ref, reg_ref, ps_ref, cnt_ref):
    i = pl.program_id(0)

    @pl.when(i == 0)
    def _():
        ps_ref[...] = jnp.zeros_like(ps_ref)
        cnt_ref[...] = jnp.zeros_like(cnt_ref)

    dis = dis_ref[...][:, 0:1]
    h = dis * (a0_ref[...] + a1_ref[...] + z_ref[...]) + b_ref[...]
    gids = lax.broadcasted_iota(jnp.int32, (_R, _NG), 1).astype(jnp.float32)
    oh = (bat_ref[...][:, 0:1] == gids).astype(jnp.float32)
    ps_ref[...] += lax.dot_general(oh, h, (((0,), (0,)), ((), ())),
                                   preferred_element_type=jnp.float32,
                                   precision=lax.Precision.HIGHEST)
    ones = jnp.ones((_R, _DEGW), jnp.float32)
    cnt_ref[...] += lax.dot_general(oh, ones, (((0,), (0,)), ((), ())),
                                    preferred_element_type=jnp.float32,
                                    precision=lax.Precision.HIGHEST)

    @pl.when(i == _GRID - 1)
    def _():
        pooled = ps_ref[...] / jnp.maximum(cnt_ref[...][:, 0:1], 1.0)
        pooled_ref[...] = pooled
        cls_ref[...] = _dot(pooled, wc_ref[...]) + bc_ref[...]
        reg_ref[...] = _dot(pooled, wr_ref[...]) + br_ref[...]


_tc_final = pl.pallas_call(
    _final_body,
    grid=(_GRID,),
    in_specs=[pl.BlockSpec((_R, _D), lambda i: (i, 0)),
              pl.BlockSpec((_R, _D), lambda i: (i, 0)),
              pl.BlockSpec((_R, _D), lambda i: (i, 0)),
              pl.BlockSpec((_R, _DEGW), lambda i: (i, 0)),
              pl.BlockSpec((1, _D), lambda i: (0, 0)),
              pl.BlockSpec((_R, _DEGW), lambda i: (i, 0)),
              pl.BlockSpec((_D, 16), lambda i: (0, 0)),
              pl.BlockSpec((1, 16), lambda i: (0, 0)),
              pl.BlockSpec((_D, 4), lambda i: (0, 0)),
              pl.BlockSpec((1, 4), lambda i: (0, 0))],
    out_specs=[pl.BlockSpec((_NG, _D), lambda i: (0, 0)),
               pl.BlockSpec((_NG, 16), lambda i: (0, 0)),
               pl.BlockSpec((_NG, 4), lambda i: (0, 0))],
    out_shape=[jax.ShapeDtypeStruct((_NG, _D), jnp.float32),
               jax.ShapeDtypeStruct((_NG, 16), jnp.float32),
               jax.ShapeDtypeStruct((_NG, 4), jnp.float32)],
    scratch_shapes=[pltpu.VMEM((_NG, _D), jnp.float32),
                    pltpu.VMEM((_NG, _DEGW), jnp.float32)],
)


def kernel(x, edge_index, batch, W1, b1, W2, b2, W3, b3, Wc, bc, Wr, br):
    src = edge_index[0]
    dst = edge_index[1]
    pad = _EP - _E
    ar = jnp.arange(pad, dtype=jnp.int32)
    srcp = jnp.concatenate([src, (ar * 8) % _N]).reshape(_NW, _CPT, _CHUNK)
    dstp = jnp.concatenate([dst, _N + (ar % (_NPAD - _N))]
                           ).reshape(_NW, _CPT, _CHUNK)

    zeros128 = jnp.zeros((_NPAD, _D), jnp.float32)
    ones_c = jnp.ones((_CHUNK, _D), jnp.float32)
    bat16 = jnp.broadcast_to(
        batch.astype(jnp.float32)[:, None], (_N, _DEGW))

    degp = _sc_deg(dstp, zeros128, ones_c)
    y1 = _tc_mm(x, W1)
    dis16, z1 = _tc_prep(degp[0, :_N], degp[1, :_N], y1)

    a1 = _sc_msg(z1, srcp, dstp, zeros128)
    z2 = _tc_layer(a1[0, :_N], a1[1, :_N], z1, dis16, b1.reshape(1, _D), W2)
    a2 = _sc_msg(z2, srcp, dstp, zeros128)
    z3 = _tc_layer(a2[0, :_N], a2[1, :_N], z2, dis16, b2.reshape(1, _D), W3)
    a3 = _sc_msg(z3, srcp, dstp, zeros128)

    pooled, cls_o, reg_o = _tc_final(
        a3[0, :_N], a3[1, :_N], z3, dis16, b3.reshape(1, _D), bat16,
        Wc, bc.reshape(1, 16), Wr, br.reshape(1, 4))
    return pooled, cls_o, reg_o
